# Initial kernel scaffold; baseline (speedup 1.0000x reference)
import jax
import jax.numpy as jnp
from jax.experimental import pallas as pl


def kernel(x, pe_table):
    raise NotImplementedError("write your pallas kernel here")


# speedup vs baseline: 1.6923x; 1.6923x over previous
import jax
import jax.numpy as jnp
from jax.experimental import pallas as pl


def _add_body(x_ref, pe_ref, out_ref):
    out_ref[...] = x_ref[...] + pe_ref[...][None, :, :]


def kernel(x, pe_table):
    B, S, D = x.shape
    BS = 512
    grid = (S // BS, B)
    return pl.pallas_call(
        _add_body,
        grid=grid,
        in_specs=[
            pl.BlockSpec((1, BS, D), lambda s, b: (b, s, 0)),
            pl.BlockSpec((BS, D), lambda s, b: (s, 0)),
        ],
        out_specs=pl.BlockSpec((1, BS, D), lambda s, b: (b, s, 0)),
        out_shape=jax.ShapeDtypeStruct((B, S, D), x.dtype),
    )(x, pe_table)
